# baseline (speedup 1.0000x reference)
import functools

import jax
import jax.numpy as jnp
from jax import lax
from jax.experimental import pallas as pl
from jax.experimental.pallas import tpu as pltpu
from jax.experimental.pallas import tpu_sc as plsc

_HID = 768
_NSL = _HID // 16
_NW = 32
_TPW = 128
_G = 8
_NCHUNK = _TPW // _G
_HALF = _NW * _TPW
_EPSILON = 1e-12


def _body(half, tok_hbm, idx6_hbm, tokemb, pos1d, px, py, ph, pw,
          out_hbm,
          tokidx, xi1, yi1, xi2, yi2, dyi, dxi,
          buf, posc, obuf, insem0, insem1, outsem0, outsem1):
    cid = lax.axis_index("c")
    sid = lax.axis_index("s")
    wid = sid * 2 + cid
    tbase = half * _HALF + wid * _TPW
    obase = wid * _TPW
    sbase = (wid % 16) * _TPW

    insems = (insem0, insem1)
    outsems = (outsem0, outsem1)

    pltpu.sync_copy(tok_hbm.at[pl.ds(tbase, _TPW)], tokidx)
    pltpu.sync_copy(idx6_hbm.at[0, pl.ds(tbase, _TPW)], xi1)
    pltpu.sync_copy(idx6_hbm.at[1, pl.ds(tbase, _TPW)], yi1)
    pltpu.sync_copy(idx6_hbm.at[2, pl.ds(tbase, _TPW)], xi2)
    pltpu.sync_copy(idx6_hbm.at[3, pl.ds(tbase, _TPW)], yi2)
    pltpu.sync_copy(idx6_hbm.at[4, pl.ds(tbase, _TPW)], dyi)
    pltpu.sync_copy(idx6_hbm.at[5, pl.ds(tbase, _TPW)], dxi)

    def in_copies(c, p):
        co = pl.ds(c * _G, _G)
        sem = insems[p]
        return [
            pltpu.make_async_copy(tokemb.at[tokidx.at[co]], buf.at[p, 0],
                                  sem),
            pltpu.make_async_copy(px.at[xi1.at[co]], buf.at[p, 1], sem),
            pltpu.make_async_copy(py.at[yi1.at[co]], buf.at[p, 2], sem),
            pltpu.make_async_copy(px.at[xi2.at[co]], buf.at[p, 3], sem),
            pltpu.make_async_copy(py.at[yi2.at[co]], buf.at[p, 4], sem),
            pltpu.make_async_copy(ph.at[dyi.at[co]], buf.at[p, 5], sem),
            pltpu.make_async_copy(pw.at[dxi.at[co]], buf.at[p, 6], sem),
            pltpu.make_async_copy(pos1d.at[pl.ds(sbase + c * _G, _G)],
                                  posc.at[p], sem),
        ]

    def issue_in(c, p):
        for d in in_copies(c, p):
            d.start()

    def wait_in(c, p):
        for d in in_copies(c, p):
            d.wait()

    def out_copy(c, p):
        return pltpu.make_async_copy(
            obuf.at[p], out_hbm.at[pl.ds(obase + c * _G, _G)], outsems[p])

    issue_in(0, 0)

    def outer(i, _):
        for b in range(2):
            c = 2 * i + b
            p = b
            cn = lax.rem(c + 1, _NCHUNK)
            issue_in(cn, 1 - p)
            wait_in(c, p)

            @pl.when(i >= 1)
            def _():
                out_copy(c, p).wait()

            for t in range(_G):
                def one(o, t=t, p=p):
                    a0 = buf[p, 0, t, o] + buf[p, 1, t, o]
                    a1 = buf[p, 2, t, o] + buf[p, 3, t, o]
                    a2 = buf[p, 4, t, o] + buf[p, 5, t, o]
                    a3 = buf[p, 6, t, o] + posc[p, t, o]
                    obuf[p, t, o] = (a0 + a1) + (a2 + a3)

                def p_sum(j, _, t=t, p=p):
                    one(pl.ds(j * 32, 16))
                    one(pl.ds(j * 32 + 16, 16))
                    return 0

                lax.fori_loop(0, _NSL // 2, p_sum, 0, unroll=4)
            out_copy(c, p).start()
        return 0

    lax.fori_loop(0, _NCHUNK // 2, outer, 0)

    wait_in(0, 0)
    out_copy(_NCHUNK - 2, 0).wait()
    out_copy(_NCHUNK - 1, 1).wait()


def _ln_body(x_ref, g_ref, b_ref, o_ref):
    x = x_ref[...]
    u = jnp.mean(x, axis=-1, keepdims=True)
    d = x - u
    v = jnp.mean(d * d, axis=-1, keepdims=True)
    r = lax.rsqrt(v + _EPSILON)
    o_ref[...] = d * r * g_ref[...] + b_ref[...]


_LN_BLK = 1024


def _layernorm_tc(x, gamma, beta):
    n = x.shape[0]
    return pl.pallas_call(
        _ln_body,
        grid=(n // _LN_BLK,),
        in_specs=[
            pl.BlockSpec((_LN_BLK, _HID), lambda i: (i, 0)),
            pl.BlockSpec((1, _HID), lambda i: (0, 0)),
            pl.BlockSpec((1, _HID), lambda i: (0, 0)),
        ],
        out_specs=pl.BlockSpec((_LN_BLK, _HID), lambda i: (i, 0)),
        out_shape=jax.ShapeDtypeStruct((n, _HID), jnp.float32),
    )(x, gamma.reshape(1, _HID), beta.reshape(1, _HID))


@jax.jit
def _emb_ln(tok_flat, idx6, tok_emb, pos1d, px, py, ph, pw, gamma, beta):
    mesh = plsc.VectorSubcoreMesh(core_axis_name="c", subcore_axis_name="s")
    scratch = [
        pltpu.VMEM((_TPW,), jnp.int32),
        pltpu.VMEM((_TPW,), jnp.int32),
        pltpu.VMEM((_TPW,), jnp.int32),
        pltpu.VMEM((_TPW,), jnp.int32),
        pltpu.VMEM((_TPW,), jnp.int32),
        pltpu.VMEM((_TPW,), jnp.int32),
        pltpu.VMEM((_TPW,), jnp.int32),
        pltpu.VMEM((2, 7, _G, _HID), jnp.float32),
        pltpu.VMEM((2, _G, _HID), jnp.float32),
        pltpu.VMEM((2, _G, _HID), jnp.float32),
        pltpu.SemaphoreType.DMA,
        pltpu.SemaphoreType.DMA,
        pltpu.SemaphoreType.DMA,
        pltpu.SemaphoreType.DMA,
    ]
    halves = []
    for h in range(2):
        f = pl.kernel(
            functools.partial(_body, h),
            mesh=mesh,
            out_type=jax.ShapeDtypeStruct((_HALF, _HID), jnp.float32),
            scratch_types=scratch,
        )
        halves.append(f(tok_flat, idx6, tok_emb, pos1d, px, py, ph, pw))
    return jnp.concatenate(
        [_layernorm_tc(s, gamma, beta) for s in halves], axis=0)


def kernel(token_ids, bbox, tok_emb, type_emb, size_emb, dir_emb, pos1d,
           pos2d_x, pos2d_y, pos2d_h, pos2d_w, gamma, beta):
    B, S = token_ids.shape
    tok_flat = token_ids.reshape(-1).astype(jnp.int32)
    bb = bbox.reshape(-1, 4).astype(jnp.int32)
    x1, y1, x2, y2 = bb[:, 0], bb[:, 1], bb[:, 2], bb[:, 3]
    idx6 = jnp.stack([x1, y1, x2, y2, y2 - y1, x2 - x1])
    const_row = type_emb[0] + size_emb[0] + dir_emb[0]
    pos1dc = pos1d + const_row[None, :]
    out = _emb_ln(tok_flat, idx6, tok_emb, pos1dc,
                  pos2d_x, pos2d_y, pos2d_h, pos2d_w, gamma, beta)
    return out.reshape(B, S, _HID)


# speedup vs baseline: 1.1950x; 1.1950x over previous
import functools

import jax
import jax.numpy as jnp
from jax import lax
from jax.experimental import pallas as pl
from jax.experimental.pallas import tpu as pltpu
from jax.experimental.pallas import tpu_sc as plsc

_HID = 768
_NSL = _HID // 16
_NW = 32
_TPW = 256
_G = 8
_NCHUNK = _TPW // _G
_EPSILON = 1e-12


def _body(tok_hbm, idx6_hbm, tokemb, pos1d, px, py, ph, pw,
          out_hbm,
          tokidx, xi1, yi1, xi2, yi2, dyi, dxi,
          buf, posc, obuf, insem0, insem1, outsem0, outsem1):
    cid = lax.axis_index("c")
    sid = lax.axis_index("s")
    wid = sid * 2 + cid
    tbase = wid * _TPW
    sbase = (wid % 8) * _TPW

    insems = (insem0, insem1)
    outsems = (outsem0, outsem1)

    pltpu.sync_copy(tok_hbm.at[pl.ds(tbase, _TPW)], tokidx)
    pltpu.sync_copy(idx6_hbm.at[0, pl.ds(tbase, _TPW)], xi1)
    pltpu.sync_copy(idx6_hbm.at[1, pl.ds(tbase, _TPW)], yi1)
    pltpu.sync_copy(idx6_hbm.at[2, pl.ds(tbase, _TPW)], xi2)
    pltpu.sync_copy(idx6_hbm.at[3, pl.ds(tbase, _TPW)], yi2)
    pltpu.sync_copy(idx6_hbm.at[4, pl.ds(tbase, _TPW)], dyi)
    pltpu.sync_copy(idx6_hbm.at[5, pl.ds(tbase, _TPW)], dxi)

    def in_copies(c, p):
        co = pl.ds(c * _G, _G)
        sem = insems[p]
        return [
            pltpu.make_async_copy(tokemb.at[tokidx.at[co]], buf.at[p, 0],
                                  sem),
            pltpu.make_async_copy(px.at[xi1.at[co]], buf.at[p, 1], sem),
            pltpu.make_async_copy(py.at[yi1.at[co]], buf.at[p, 2], sem),
            pltpu.make_async_copy(px.at[xi2.at[co]], buf.at[p, 3], sem),
            pltpu.make_async_copy(py.at[yi2.at[co]], buf.at[p, 4], sem),
            pltpu.make_async_copy(ph.at[dyi.at[co]], buf.at[p, 5], sem),
            pltpu.make_async_copy(pw.at[dxi.at[co]], buf.at[p, 6], sem),
            pltpu.make_async_copy(pos1d.at[pl.ds(sbase + c * _G, _G)],
                                  posc.at[p], sem),
        ]

    def issue_in(c, p):
        for d in in_copies(c, p):
            d.start()

    def wait_in(c, p):
        for d in in_copies(c, p):
            d.wait()

    def out_copy(c, p):
        return pltpu.make_async_copy(
            obuf.at[p], out_hbm.at[pl.ds(tbase + c * _G, _G)], outsems[p])

    issue_in(0, 0)

    def outer(i, _):
        for b in range(2):
            c = 2 * i + b
            p = b
            @pl.when(c + 1 < _NCHUNK)
            def _():
                issue_in(c + 1, 1 - p)

            wait_in(c, p)

            @pl.when(i >= 1)
            def _():
                out_copy(c, p).wait()

            for t in range(_G):
                def one(o, t=t, p=p):
                    a0 = buf[p, 0, t, o] + buf[p, 1, t, o]
                    a1 = buf[p, 2, t, o] + buf[p, 3, t, o]
                    a2 = buf[p, 4, t, o] + buf[p, 5, t, o]
                    a3 = buf[p, 6, t, o] + posc[p, t, o]
                    obuf[p, t, o] = (a0 + a1) + (a2 + a3)

                def p_sum(j, _, t=t, p=p):
                    one(pl.ds(j * 32, 16))
                    one(pl.ds(j * 32 + 16, 16))
                    return 0

                lax.fori_loop(0, _NSL // 2, p_sum, 0, unroll=4)
            out_copy(c, p).start()
        return 0

    lax.fori_loop(0, _NCHUNK // 2, outer, 0)

    out_copy(_NCHUNK - 2, 0).wait()
    out_copy(_NCHUNK - 1, 1).wait()


def _ln_body(x_ref, g_ref, b_ref, o_ref):
    x = x_ref[...]
    u = jnp.mean(x, axis=-1, keepdims=True)
    d = x - u
    v = jnp.mean(d * d, axis=-1, keepdims=True)
    r = lax.rsqrt(v + _EPSILON)
    o_ref[...] = d * r * g_ref[...] + b_ref[...]


_LN_BLK = 1024


def _layernorm_tc(x, gamma, beta):
    n = x.shape[0]
    return pl.pallas_call(
        _ln_body,
        grid=(n // _LN_BLK,),
        in_specs=[
            pl.BlockSpec((_LN_BLK, _HID), lambda i: (i, 0)),
            pl.BlockSpec((1, _HID), lambda i: (0, 0)),
            pl.BlockSpec((1, _HID), lambda i: (0, 0)),
        ],
        out_specs=pl.BlockSpec((_LN_BLK, _HID), lambda i: (i, 0)),
        out_shape=jax.ShapeDtypeStruct((n, _HID), jnp.float32),
    )(x, gamma.reshape(1, _HID), beta.reshape(1, _HID))


@jax.jit
def _emb_ln(tok_flat, idx6, tok_emb, pos1d, px, py, ph, pw, gamma, beta):
    mesh = plsc.VectorSubcoreMesh(core_axis_name="c", subcore_axis_name="s")
    f = pl.kernel(
        _body,
        mesh=mesh,
        out_type=jax.ShapeDtypeStruct((_NW * _TPW, _HID), jnp.float32),
        scratch_types=[
            pltpu.VMEM((_TPW,), jnp.int32),
            pltpu.VMEM((_TPW,), jnp.int32),
            pltpu.VMEM((_TPW,), jnp.int32),
            pltpu.VMEM((_TPW,), jnp.int32),
            pltpu.VMEM((_TPW,), jnp.int32),
            pltpu.VMEM((_TPW,), jnp.int32),
            pltpu.VMEM((_TPW,), jnp.int32),
            pltpu.VMEM((2, 7, _G, _HID), jnp.float32),
            pltpu.VMEM((2, _G, _HID), jnp.float32),
            pltpu.VMEM((2, _G, _HID), jnp.float32),
            pltpu.SemaphoreType.DMA,
            pltpu.SemaphoreType.DMA,
            pltpu.SemaphoreType.DMA,
            pltpu.SemaphoreType.DMA,
        ],
    )
    summed = f(tok_flat, idx6, tok_emb, pos1d, px, py, ph, pw)
    return _layernorm_tc(summed, gamma, beta)


def kernel(token_ids, bbox, tok_emb, type_emb, size_emb, dir_emb, pos1d,
           pos2d_x, pos2d_y, pos2d_h, pos2d_w, gamma, beta):
    B, S = token_ids.shape
    tok_flat = token_ids.reshape(-1).astype(jnp.int32)
    bb = bbox.reshape(-1, 4).astype(jnp.int32)
    x1, y1, x2, y2 = bb[:, 0], bb[:, 1], bb[:, 2], bb[:, 3]
    idx6 = jnp.stack([x1, y1, x2, y2, y2 - y1, x2 - x1])
    const_row = type_emb[0] + size_emb[0] + dir_emb[0]
    pos1dc = pos1d + const_row[None, :]
    out = _emb_ln(tok_flat, idx6, tok_emb, pos1dc,
                  pos2d_x, pos2d_y, pos2d_h, pos2d_w, gamma, beta)
    return out.reshape(B, S, _HID)
